# baseline (speedup 1.0000x reference)
import functools

import jax
import jax.numpy as jnp
from jax import lax
from jax.experimental import pallas as pl
from jax.experimental.pallas import tpu as pltpu
from jax.experimental.pallas import tpu_sc as plsc

NC = 2
NS = 16
CHUNK = 128

F32 = jnp.float32


def _node_body(p_ref, wpp_ref, bpp_ref, wpi1_ref, bpi1_ref, a_ref, b_ref):
    d = p_ref.shape[1]
    p1 = jnp.tanh(
        jnp.dot(p_ref[...], wpp_ref[...], preferred_element_type=F32)
        + bpp_ref[...]
    )
    a_ref[...] = (jnp.dot(p1, wpi1_ref[:d, :], preferred_element_type=F32)
                  + bpi1_ref[...])
    b_ref[...] = jnp.dot(p1, wpi1_ref[d:, :], preferred_element_type=F32)


def _node_call(p, W_pp, b_pp, W_pi1, b_pi1):
    n, d = p.shape
    h = W_pi1.shape[1]
    return pl.pallas_call(
        _node_body,
        out_shape=(
            jax.ShapeDtypeStruct((n, h), F32),
            jax.ShapeDtypeStruct((n, h), F32),
        ),
    )(p, W_pp, b_pp.reshape(1, d), W_pi1, b_pi1.reshape(1, h))


def _gather_body(n_chunks_w, a_hbm, b_hbm, ii_hbm, jj_hbm, x_hbm,
                 idxi_v, idxj_v, rowsa_v, sema, semb):
    wid = lax.axis_index("s") * NC + lax.axis_index("c")
    base = wid * n_chunks_w * CHUNK

    def body(c, carry):
        off = base + c * CHUNK
        pltpu.sync_copy(ii_hbm.at[pl.ds(off, CHUNK)], idxi_v)
        pltpu.sync_copy(jj_hbm.at[pl.ds(off, CHUNK)], idxj_v)
        cpa = pltpu.async_copy(a_hbm.at[idxi_v], rowsa_v, sema)
        cpa.wait()
        cpb = pltpu.async_copy(b_hbm.at[idxj_v], rowsa_v, semb, add=True)
        cpb.wait()
        pltpu.sync_copy(rowsa_v, x_hbm.at[pl.ds(off, CHUNK)])
        return carry

    lax.fori_loop(0, n_chunks_w, body, 0)


def _gather_call(A, B, idx_i_p, idx_j_p):
    h = A.shape[1]
    ep = idx_i_p.shape[0]
    n_chunks_w = ep // (NC * NS * CHUNK)
    mesh = plsc.VectorSubcoreMesh(
        core_axis_name="c", subcore_axis_name="s", num_cores=NC,
        num_subcores=NS)
    kern = functools.partial(
        pl.kernel,
        out_type=jax.ShapeDtypeStruct((ep, h), F32),
        mesh=mesh,
        scratch_types=[
            pltpu.VMEM((CHUNK,), jnp.int32),
            pltpu.VMEM((CHUNK,), jnp.int32),
            pltpu.VMEM((CHUNK, h), F32),
            pltpu.SemaphoreType.DMA,
            pltpu.SemaphoreType.DMA,
        ],
        compiler_params=pltpu.CompilerParams(use_tc_tiling_on_sc=False),
    )(functools.partial(_gather_body, n_chunks_w))
    return kern(A, B, idx_i_p, idx_j_p)


def _edge_body(e_valid, be, nb, x_ref, basis_ref, w2p_ref, wii_ref,
               bii_ref, out_ref):
    h = x_ref.shape[1]
    t = jnp.dot(x_ref[...], w2p_ref[...],
                preferred_element_type=F32)
    y = t[:, 0:h] * basis_ref[:, 0:1]
    for c in range(1, nb):
        y = y + t[:, c * h:(c + 1) * h] * basis_ref[:, c:c + 1]
    i1 = jnp.tanh(jnp.dot(y, wii_ref[...], preferred_element_type=F32)
                  + bii_ref[...])
    row = pl.program_id(0) * be + lax.broadcasted_iota(jnp.int32, i1.shape, 0)
    out_ref[...] = jnp.where(row < e_valid, i1, 0.0)


def _edge_call(x, basis, W2perm, W_ii, b_ii, e_valid, row0):
    ck, h = x.shape
    e, nb = basis.shape
    d = W_ii.shape[1]
    be = 512
    grid = ck // be
    cb = row0 // be
    last_full = e // be - 1
    return pl.pallas_call(
        functools.partial(_edge_body, e_valid - row0, be, nb),
        grid=(grid,),
        in_specs=[
            pl.BlockSpec((be, h), lambda g: (g, 0)),
            pl.BlockSpec((be, nb),
                         lambda g: (jnp.minimum(cb + g, last_full), 0)),
            pl.BlockSpec((h, nb * h), lambda g: (0, 0)),
            pl.BlockSpec((h, d), lambda g: (0, 0)),
            pl.BlockSpec((1, d), lambda g: (0, 0)),
        ],
        out_specs=pl.BlockSpec((be, d), lambda g: (g, 0)),
        out_shape=jax.ShapeDtypeStruct((ck, d), F32),
        compiler_params=pltpu.CompilerParams(
            dimension_semantics=("arbitrary",)),
    )(x, basis, W2perm, W_ii, b_ii.reshape(1, d))


def _scatter_body(n_chunks, n_pad, i1_hbm, ii_hbm, zeros_hbm, parts_hbm,
                  idx_v, rows_v, slab_v, shared):
    cid = lax.axis_index("c")
    sid = lax.axis_index("s")
    slab = n_pad // NS
    dh = rows_v.shape[1]

    pltpu.sync_copy(zeros_hbm.at[pl.ds(sid * slab, slab)], slab_v)
    pltpu.sync_copy(slab_v, shared.at[pl.ds(sid * slab, slab)])
    plsc.subcore_barrier()

    base = sid * n_chunks * CHUNK

    def body(c, carry):
        off = base + c * CHUNK
        pltpu.sync_copy(ii_hbm.at[pl.ds(off, CHUNK)], idx_v)
        pltpu.sync_copy(i1_hbm.at[pl.ds(off, CHUNK), pl.ds(cid * dh, dh)],
                        rows_v)
        pltpu.sync_copy(rows_v, shared.at[idx_v], add=True)
        return carry

    lax.fori_loop(0, n_chunks, body, 0)
    plsc.subcore_barrier()

    pltpu.sync_copy(shared.at[pl.ds(sid * slab, slab)], slab_v)
    pltpu.sync_copy(slab_v, parts_hbm.at[pl.ds(cid * n_pad + sid * slab,
                                               slab)])


def _scatter_call(i1, idx_i_p, zeros_nd):
    ep, d = i1.shape
    n_pad, dh = zeros_nd.shape
    n_chunks = ep // (NS * CHUNK)
    mesh = plsc.VectorSubcoreMesh(
        core_axis_name="c", subcore_axis_name="s", num_cores=NC,
        num_subcores=NS)
    kern = functools.partial(
        pl.kernel,
        out_type=jax.ShapeDtypeStruct((NC * n_pad, dh), F32),
        mesh=mesh,
        scratch_types=[
            pltpu.VMEM((CHUNK,), jnp.int32),
            pltpu.VMEM((CHUNK, dh), F32),
            pltpu.VMEM((n_pad // NS, dh), F32),
            pltpu.VMEM_SHARED((n_pad, dh), F32),
        ],
        compiler_params=pltpu.CompilerParams(use_tc_tiling_on_sc=False),
    )(functools.partial(_scatter_body, n_chunks, n_pad))
    return kern(i1, idx_i_p, zeros_nd)


def _combine_body(p_ref, *refs):
    parts_refs = refs[:-1]
    out_ref = refs[-1]
    dh = parts_refs[0].shape[2]
    k = len(parts_refs) // 2
    lo = p_ref[:, :dh]
    hi = p_ref[:, dh:]
    for pr in parts_refs[:k]:
        lo = lo + pr[0]
    for pr in parts_refs[k:]:
        hi = hi + pr[0]
    out_ref[:, :dh] = lo
    out_ref[:, dh:] = hi


def _combine_call(p, parts_list):
    n, d = p.shape
    n_pad = parts_list[0].shape[0] // NC
    dh = parts_list[0].shape[1]
    bn = n_pad // 8
    grid = (n + bn - 1) // bn
    parts3 = [q.reshape(NC, n_pad, dh) for q in parts_list]
    in_specs = [pl.BlockSpec((bn, d), lambda g: (g, 0))]
    in_specs += [pl.BlockSpec((1, bn, dh), lambda g: (0, g, 0))
                 for _ in parts3]
    in_specs += [pl.BlockSpec((1, bn, dh), lambda g: (1, g, 0))
                 for _ in parts3]
    return pl.pallas_call(
        _combine_body,
        grid=(grid,),
        in_specs=in_specs,
        out_specs=pl.BlockSpec((bn, d), lambda g: (g, 0)),
        out_shape=jax.ShapeDtypeStruct((n, d), F32),
    )(p, *parts3, *parts3)


def kernel(p, idx_i, idx_j, basis, W_pp, b_pp, W_pi1, b_pi1, W_pi2, W_ii,
           b_ii):
    n, d = p.shape
    e = idx_i.shape[0]
    h = W_pi1.shape[1]
    nb = basis.shape[1]

    K = 8
    align = K * NC * NS * CHUNK
    ep = ((e + align - 1) // align) * align
    ck = ep // K
    pad = ep - e
    idx_i_p = jnp.pad(idx_i, (0, pad))
    idx_j_p = jnp.pad(idx_j, (0, pad))

    W2perm = W_pi2.reshape(h, h, nb).transpose(0, 2, 1).reshape(h, nb * h)

    nalign = NS * 8
    n_pad = ((n + nalign - 1) // nalign) * nalign
    zeros_nd = jnp.zeros((n_pad, d // NC), F32)

    A, B = _node_call(p, W_pp, b_pp, W_pi1, b_pi1)
    xs = []
    for k in range(K):
        sl = slice(k * ck, (k + 1) * ck)
        xs.append(_gather_call(A, B, idx_i_p[sl], idx_j_p[sl]))
    parts_list = []
    for k in range(K):
        sl = slice(k * ck, (k + 1) * ck)
        i1_k = _edge_call(xs[k], basis, W2perm, W_ii, b_ii, e, k * ck)
        parts_list.append(_scatter_call(i1_k, idx_i_p[sl], zeros_nd))
    return _combine_call(p, parts_list)


# speedup vs baseline: 1.0167x; 1.0167x over previous
import functools

import jax
import jax.numpy as jnp
from jax import lax
from jax.experimental import pallas as pl
from jax.experimental.pallas import tpu as pltpu
from jax.experimental.pallas import tpu_sc as plsc

NC = 2
NS = 16
CHUNK = 128

F32 = jnp.float32


def _node_body(p_ref, wpp_ref, bpp_ref, wpi1_ref, bpi1_ref, a_ref, b_ref):
    d = p_ref.shape[1]
    p1 = jnp.tanh(
        jnp.dot(p_ref[...], wpp_ref[...], preferred_element_type=F32)
        + bpp_ref[...]
    )
    a_ref[...] = (jnp.dot(p1, wpi1_ref[:d, :], preferred_element_type=F32)
                  + bpi1_ref[...])
    b_ref[...] = jnp.dot(p1, wpi1_ref[d:, :], preferred_element_type=F32)


def _node_call(p, W_pp, b_pp, W_pi1, b_pi1):
    n, d = p.shape
    h = W_pi1.shape[1]
    return pl.pallas_call(
        _node_body,
        out_shape=(
            jax.ShapeDtypeStruct((n, h), F32),
            jax.ShapeDtypeStruct((n, h), F32),
        ),
    )(p, W_pp, b_pp.reshape(1, d), W_pi1, b_pi1.reshape(1, h))


def _gather_body(n_chunks_w, a_hbm, b_hbm, ii_hbm, jj_hbm, x_hbm,
                 idxi_v, idxj_v, rowsa_v, sema, semb):
    wid = lax.axis_index("s") * NC + lax.axis_index("c")
    base = wid * n_chunks_w * CHUNK

    def body(c, carry):
        off = base + c * CHUNK
        pltpu.sync_copy(ii_hbm.at[pl.ds(off, CHUNK)], idxi_v)
        pltpu.sync_copy(jj_hbm.at[pl.ds(off, CHUNK)], idxj_v)
        cpa = pltpu.async_copy(a_hbm.at[idxi_v], rowsa_v, sema)
        cpa.wait()
        cpb = pltpu.async_copy(b_hbm.at[idxj_v], rowsa_v, semb, add=True)
        cpb.wait()
        pltpu.sync_copy(rowsa_v, x_hbm.at[pl.ds(off, CHUNK)])
        return carry

    lax.fori_loop(0, n_chunks_w, body, 0)


def _gather_call(A, B, idx_i_p, idx_j_p):
    h = A.shape[1]
    ep = idx_i_p.shape[0]
    n_chunks_w = ep // (NC * NS * CHUNK)
    mesh = plsc.VectorSubcoreMesh(
        core_axis_name="c", subcore_axis_name="s", num_cores=NC,
        num_subcores=NS)
    kern = functools.partial(
        pl.kernel,
        out_type=jax.ShapeDtypeStruct((ep, h), F32),
        mesh=mesh,
        scratch_types=[
            pltpu.VMEM((CHUNK,), jnp.int32),
            pltpu.VMEM((CHUNK,), jnp.int32),
            pltpu.VMEM((CHUNK, h), F32),
            pltpu.SemaphoreType.DMA,
            pltpu.SemaphoreType.DMA,
        ],
        compiler_params=pltpu.CompilerParams(use_tc_tiling_on_sc=False),
    )(functools.partial(_gather_body, n_chunks_w))
    return kern(A, B, idx_i_p, idx_j_p)


def _edge_body(e_valid, be, nb, x_ref, basis_ref, w2p_ref, wii_ref,
               bii_ref, out_ref):
    h = x_ref.shape[1]
    t = jnp.dot(x_ref[...], w2p_ref[...],
                preferred_element_type=F32)
    y = t[:, 0:h] * basis_ref[:, 0:1]
    for c in range(1, nb):
        y = y + t[:, c * h:(c + 1) * h] * basis_ref[:, c:c + 1]
    i1 = jnp.tanh(jnp.dot(y, wii_ref[...], preferred_element_type=F32)
                  + bii_ref[...])
    row = pl.program_id(0) * be + lax.broadcasted_iota(jnp.int32, i1.shape, 0)
    out_ref[...] = jnp.where(row < e_valid, i1, 0.0)


def _edge_call(x, basis, W2perm, W_ii, b_ii, e_valid, row0):
    ck, h = x.shape
    e, nb = basis.shape
    d = W_ii.shape[1]
    be = 512
    grid = ck // be
    cb = row0 // be
    last_full = e // be - 1
    return pl.pallas_call(
        functools.partial(_edge_body, e_valid - row0, be, nb),
        grid=(grid,),
        in_specs=[
            pl.BlockSpec((be, h), lambda g: (g, 0)),
            pl.BlockSpec((be, nb),
                         lambda g: (jnp.minimum(cb + g, last_full), 0)),
            pl.BlockSpec((h, nb * h), lambda g: (0, 0)),
            pl.BlockSpec((h, d), lambda g: (0, 0)),
            pl.BlockSpec((1, d), lambda g: (0, 0)),
        ],
        out_specs=pl.BlockSpec((be, d), lambda g: (g, 0)),
        out_shape=jax.ShapeDtypeStruct((ck, d), F32),
        compiler_params=pltpu.CompilerParams(
            dimension_semantics=("arbitrary",)),
    )(x, basis, W2perm, W_ii, b_ii.reshape(1, d))


def _scatter_body(n_chunks, n_pad, i1_hbm, ii_hbm, zeros_hbm, parts_hbm,
                  idx_v, rows_v, slab_v, shared):
    cid = lax.axis_index("c")
    sid = lax.axis_index("s")
    slab = n_pad // NS
    dh = rows_v.shape[1]

    pltpu.sync_copy(zeros_hbm.at[pl.ds(sid * slab, slab)], slab_v)
    pltpu.sync_copy(slab_v, shared.at[pl.ds(sid * slab, slab)])
    plsc.subcore_barrier()

    base = sid * n_chunks * CHUNK

    def body(c, carry):
        off = base + c * CHUNK
        pltpu.sync_copy(ii_hbm.at[pl.ds(off, CHUNK)], idx_v)
        pltpu.sync_copy(i1_hbm.at[pl.ds(off, CHUNK), pl.ds(cid * dh, dh)],
                        rows_v)
        pltpu.sync_copy(rows_v, shared.at[idx_v], add=True)
        return carry

    lax.fori_loop(0, n_chunks, body, 0)
    plsc.subcore_barrier()

    pltpu.sync_copy(shared.at[pl.ds(sid * slab, slab)], slab_v)
    pltpu.sync_copy(slab_v, parts_hbm.at[pl.ds(cid * n_pad + sid * slab,
                                               slab)])


def _scatter_call(i1, idx_i_p, zeros_nd):
    ep, d = i1.shape
    n_pad, dh = zeros_nd.shape
    n_chunks = ep // (NS * CHUNK)
    mesh = plsc.VectorSubcoreMesh(
        core_axis_name="c", subcore_axis_name="s", num_cores=NC,
        num_subcores=NS)
    kern = functools.partial(
        pl.kernel,
        out_type=jax.ShapeDtypeStruct((NC * n_pad, dh), F32),
        mesh=mesh,
        scratch_types=[
            pltpu.VMEM((CHUNK,), jnp.int32),
            pltpu.VMEM((CHUNK, dh), F32),
            pltpu.VMEM((n_pad // NS, dh), F32),
            pltpu.VMEM_SHARED((n_pad, dh), F32),
        ],
        compiler_params=pltpu.CompilerParams(use_tc_tiling_on_sc=False),
    )(functools.partial(_scatter_body, n_chunks, n_pad))
    return kern(i1, idx_i_p, zeros_nd)


def _combine_body(p_ref, *refs):
    parts_refs = refs[:-1]
    out_ref = refs[-1]
    dh = parts_refs[0].shape[2]
    k = len(parts_refs) // 2
    lo = p_ref[:, :dh]
    hi = p_ref[:, dh:]
    for pr in parts_refs[:k]:
        lo = lo + pr[0]
    for pr in parts_refs[k:]:
        hi = hi + pr[0]
    out_ref[:, :dh] = lo
    out_ref[:, dh:] = hi


def _combine_call(p, parts_list):
    n, d = p.shape
    n_pad = parts_list[0].shape[0] // NC
    dh = parts_list[0].shape[1]
    bn = n_pad // 8
    grid = (n + bn - 1) // bn
    parts3 = [q.reshape(NC, n_pad, dh) for q in parts_list]
    in_specs = [pl.BlockSpec((bn, d), lambda g: (g, 0))]
    in_specs += [pl.BlockSpec((1, bn, dh), lambda g: (0, g, 0))
                 for _ in parts3]
    in_specs += [pl.BlockSpec((1, bn, dh), lambda g: (1, g, 0))
                 for _ in parts3]
    return pl.pallas_call(
        _combine_body,
        grid=(grid,),
        in_specs=in_specs,
        out_specs=pl.BlockSpec((bn, d), lambda g: (g, 0)),
        out_shape=jax.ShapeDtypeStruct((n, d), F32),
    )(p, *parts3, *parts3)


def kernel(p, idx_i, idx_j, basis, W_pp, b_pp, W_pi1, b_pi1, W_pi2, W_ii,
           b_ii):
    n, d = p.shape
    e = idx_i.shape[0]
    h = W_pi1.shape[1]
    nb = basis.shape[1]

    K = 4
    align = K * NC * NS * CHUNK
    ep = ((e + align - 1) // align) * align
    ck = ep // K
    pad = ep - e
    idx_i_p = jnp.pad(idx_i, (0, pad))
    idx_j_p = jnp.pad(idx_j, (0, pad))

    W2perm = W_pi2.reshape(h, h, nb).transpose(0, 2, 1).reshape(h, nb * h)

    nalign = NS * 8
    n_pad = ((n + nalign - 1) // nalign) * nalign
    zeros_nd = jnp.zeros((n_pad, d // NC), F32)

    A, B = _node_call(p, W_pp, b_pp, W_pi1, b_pi1)
    xs = []
    for k in range(K):
        sl = slice(k * ck, (k + 1) * ck)
        xs.append(_gather_call(A, B, idx_i_p[sl], idx_j_p[sl]))
    parts_list = []
    for k in range(K):
        sl = slice(k * ck, (k + 1) * ck)
        i1_k = _edge_call(xs[k], basis, W2perm, W_ii, b_ii, e, k * ck)
        parts_list.append(_scatter_call(i1_k, idx_i_p[sl], zeros_nd))
    return _combine_call(p, parts_list)
